# baseline (speedup 1.0000x reference)
import functools

import jax
import jax.numpy as jnp
from jax import lax
from jax.experimental import pallas as pl
from jax.experimental.pallas import tpu as pltpu
from jax.experimental.pallas import tpu_sc as plsc

NUM_CORES = 2
NUM_SUBCORES = 16
NUM_WORKERS = NUM_CORES * NUM_SUBCORES


def _select_half(idx_ref, wide_ref, out_ref):
    em = out_ref.shape[1]
    parity = idx_ref[...] & 1
    out_ref[...] = jnp.where(parity == 1, wide_ref[:, em:], wide_ref[:, :em])


@jax.jit
def kernel(hyperparameters, table):
    num_emb, em_size = table.shape
    batch = hyperparameters.shape[0]
    idx = jnp.squeeze(hyperparameters, axis=1).astype(jnp.int32)
    idx2 = idx >> 1
    wide_table = table.reshape(num_emb // 2, 2 * em_size)
    b_per_w = batch // NUM_WORKERS

    mesh = plsc.VectorSubcoreMesh(core_axis_name="c", subcore_axis_name="s")

    @functools.partial(
        pl.kernel,
        mesh=mesh,
        out_type=jax.ShapeDtypeStruct((batch, 2 * em_size), jnp.float32),
        scratch_types=[
            pltpu.VMEM((b_per_w,), jnp.int32),
            pltpu.VMEM((b_per_w, 2 * em_size), jnp.float32),
            pltpu.SemaphoreType.DMA,
        ],
    )
    def emb_gather(table_hbm, idx_hbm, out_hbm, idx_v, rows_v, sem):
        wid = lax.axis_index("s") * NUM_CORES + lax.axis_index("c")
        base = wid * b_per_w
        pltpu.sync_copy(idx_hbm.at[pl.ds(base, b_per_w)], idx_v)
        pltpu.async_copy(table_hbm.at[idx_v], rows_v, sem).wait()
        pltpu.sync_copy(rows_v, out_hbm.at[pl.ds(base, b_per_w)])

    wide_rows = emb_gather(wide_table, idx2)

    return pl.pallas_call(
        _select_half,
        out_shape=jax.ShapeDtypeStruct((batch, em_size), jnp.float32),
    )(hyperparameters.astype(jnp.int32), wide_rows)


# speedup vs baseline: 1.1376x; 1.1376x over previous
import functools

import jax
import jax.numpy as jnp
from jax import lax
from jax.experimental import pallas as pl
from jax.experimental.pallas import tpu as pltpu
from jax.experimental.pallas import tpu_sc as plsc

NUM_CORES = 2
NUM_SUBCORES = 16
NUM_WORKERS = NUM_CORES * NUM_SUBCORES
LANES = 16


@jax.jit
def kernel(hyperparameters, table):
    num_emb, em_size = table.shape
    batch = hyperparameters.shape[0]
    idx = jnp.squeeze(hyperparameters, axis=1).astype(jnp.int32)
    b_per_w = batch // NUM_WORKERS

    mesh = plsc.VectorSubcoreMesh(core_axis_name="c", subcore_axis_name="s")

    @functools.partial(
        pl.kernel,
        mesh=mesh,
        out_type=jax.ShapeDtypeStruct((batch, em_size), jnp.float32),
        compiler_params=pltpu.CompilerParams(use_tc_tiling_on_sc=False),
        scratch_types=[
            pltpu.VMEM((b_per_w,), jnp.int32),
            pltpu.VMEM((b_per_w, em_size), jnp.float32),
            pltpu.SemaphoreType.DMA,
        ],
    )
    def emb_lookup(table_hbm, idx_hbm, out_hbm, idx_v, rows_v, sem):
        wid = lax.axis_index("s") * NUM_CORES + lax.axis_index("c")
        base = wid * b_per_w
        pltpu.sync_copy(idx_hbm.at[pl.ds(base, b_per_w)], idx_v)

        @pl.loop(0, b_per_w, step=LANES)
        def _(k):
            v = idx_v[pl.ds(k, LANES)]
            pltpu.async_copy(
                table_hbm.at[v], rows_v.at[pl.ds(k, LANES)], sem
            )

        pltpu.make_async_copy(
            table_hbm.at[pl.ds(0, b_per_w)],
            rows_v,
            sem,
        ).wait()
        pltpu.sync_copy(rows_v, out_hbm.at[pl.ds(base, b_per_w)])

    return emb_lookup(table, idx)
